# speedup vs baseline: 1.0111x; 1.0111x over previous
import functools

import jax
import jax.numpy as jnp
from jax.experimental import pallas as pl

N = 10000
E = 320000
F_IN = 128
D_E = 16
NUM_CLASSES = 10
H = NUM_CLASSES * 2
C = NUM_CLASSES


def _mm_body(a_ref, b_ref, o_ref):
    o_ref[...] = jnp.dot(a_ref[...], b_ref[...],
                         preferred_element_type=jnp.float32)


def _mm(a, b, bm):
    m, k = a.shape
    _, n = b.shape
    return pl.pallas_call(
        _mm_body,
        grid=(m // bm,),
        in_specs=[pl.BlockSpec((bm, k), lambda i: (i, 0)),
                  pl.BlockSpec((k, n), lambda i: (0, 0))],
        out_specs=pl.BlockSpec((bm, n), lambda i: (i, 0)),
        out_shape=jax.ShapeDtypeStruct((m, n), jnp.float32),
    )(a, b)


def _gat_layer(x, src, dst, alpha_edge, W, a_src, a_dst, bias, bm):
    h = _mm(x, W, bm)
    idx = jnp.arange(H * C)
    A_s = jnp.zeros((H * C, H), jnp.float32).at[idx, idx // C].set(a_src.reshape(-1))
    A_d = jnp.zeros((H * C, H), jnp.float32).at[idx, idx // C].set(a_dst.reshape(-1))
    alpha_src = h @ A_s
    alpha_dst = h @ A_d
    alpha = alpha_src[src] + alpha_dst[dst] + alpha_edge
    alpha = jax.nn.leaky_relu(alpha, 0.2)
    amax = jax.ops.segment_max(alpha, dst, num_segments=N)
    amax = jnp.where(jnp.isfinite(amax), amax, 0.0)
    ex = jnp.exp(alpha - amax[dst])
    denom = jax.ops.segment_sum(ex, dst, num_segments=N)
    alpha = ex / (denom[dst] + 1e-16)
    hh = h.reshape(N, H, C)
    msg = hh[src] * alpha[:, :, None]
    out = jax.ops.segment_max(msg, dst, num_segments=N)
    out = jnp.where(jnp.isfinite(out), out, 0.0)
    return out.mean(axis=1) + bias


def kernel(x, edge_index, edge_attr, W1, as1, ad1, We1, ae1, b1,
           W2, as2, ad2, We2, ae2, b2):
    src = edge_index[0]
    dst = edge_index[1]
    B1 = (We1.reshape(D_E, H, C) * ae1[None]).sum(-1)
    B2 = (We2.reshape(D_E, H, C) * ae2[None]).sum(-1)
    ae_both = _mm(edge_attr, jnp.concatenate([B1, B2], axis=1), 8000)
    h1 = _gat_layer(x, src, dst, ae_both[:, :H], W1, as1, ad1, b1, 1000)
    h1 = jax.nn.relu(h1)
    h2 = _gat_layer(h1, src, dst, ae_both[:, H:], W2, as2, ad2, b2, 1000)
    return jax.nn.log_softmax(h2, axis=1)


# speedup vs baseline: 16.7123x; 16.5280x over previous
import functools

import jax
import jax.numpy as jnp
from jax import lax
from jax.experimental import pallas as pl
from jax.experimental.pallas import tpu as pltpu
from jax.experimental.pallas import tpu_sc as plsc

N = 10000
E = 320000
F_IN = 128
D_E = 16
NUM_CLASSES = 10
H = NUM_CLASSES * 2
C = NUM_CLASSES

NC = 2
NS = 16
NW = NC * NS
NPT = 320
CH = 3200
FL = 4096
CAP = E + 2 * FL
CP = 512
CE = 256
W = 128
HD = 208

_mesh = plsc.VectorSubcoreMesh(core_axis_name="c", subcore_axis_name="s")
_sc_params = pltpu.CompilerParams(needs_layout_passes=False,
                                  use_tc_tiling_on_sc=False)


@functools.partial(
    pl.kernel, mesh=_mesh, compiler_params=_sc_params,
    out_type=(jax.ShapeDtypeStruct((NW * CAP,), jnp.int32),
              jax.ShapeDtypeStruct((NW * CAP,), jnp.int32),
              jax.ShapeDtypeStruct((NW * CAP + 16,), jnp.int32),
              jax.ShapeDtypeStruct((NW * CAP + 16,), jnp.int32),
              jax.ShapeDtypeStruct((NW * 512,), jnp.int32),
              jax.ShapeDtypeStruct((NW * 128,), jnp.int32)),
    scratch_types=[pltpu.VMEM((CH,), jnp.int32),
                   pltpu.VMEM((CH,), jnp.int32),
                   pltpu.VMEM((2 * FL + 16,), jnp.int32),
                   pltpu.VMEM((2 * FL + 16,), jnp.int32),
                   pltpu.VMEM((16,), jnp.int32),
                   pltpu.VMEM((NPT,), jnp.int32),
                   pltpu.VMEM((512,), jnp.int32),
                   pltpu.VMEM((NPT,), jnp.int32),
                   pltpu.VMEM((CP,), jnp.int32),
                   pltpu.VMEM((CP,), jnp.int32),
                   pltpu.VMEM((CP,), jnp.int32),
                   pltpu.SMEM((8,), jnp.int32),
                   pltpu.SemaphoreType.DMA],
)
def _partition(src_hbm, dst_hbm, out_pk, out_id, out_spk, out_sid,
               out_off, out_cnt,
               dstb, srcb, stg_pk, stg_id, vtmp, hv, offv, cur,
               pkb, idb, posb, sptr, sem):
    wid = lax.axis_index("s") * NC + lax.axis_index("c")
    lo = wid * NPT
    hi = jnp.minimum(lo + NPT, N)
    lane = lax.iota(jnp.int32, 16)
    ones = jnp.broadcast_to(jnp.int32(1), (16,))

    sptr[0] = 0
    sptr[1] = 0

    def hz(j, _):
        hv[pl.ds(j * 16, 16)] = jnp.broadcast_to(jnp.int32(0), (16,))
        return 0
    lax.fori_loop(0, NPT // 16, hz, 0)

    def chunk_body(ci, _):
        base = ci * CH
        pltpu.sync_copy(dst_hbm.at[pl.ds(base, CH)], dstb)
        pltpu.sync_copy(src_hbm.at[pl.ds(base, CH)], srcb)

        def vec_body(i, _):
            d = dstb[pl.ds(i * 16, 16)]
            s = srcb[pl.ds(i * 16, 16)]
            m = (d >= lo) & (d < hi)
            dl = jnp.where(m, d - lo, 0)
            pk = s * 1024 + dl
            eid = base + i * 16 + lane
            p = sptr[0]
            cs = plsc.cumsum(m.astype(jnp.int32))
            tgt = p + cs - 1
            plsc.store_scatter(stg_pk, [tgt], pk, mask=m)
            plsc.store_scatter(stg_id, [tgt], eid, mask=m)
            plsc.addupdate_scatter(hv, [dl], ones, mask=m)
            sptr[0] = p + cs[15]
            return 0

        lax.fori_loop(0, CH // 16, vec_body, 0)

        @pl.when(sptr[0] >= FL)
        def _():
            f = pl.multiple_of(sptr[1], 128)
            pltpu.sync_copy(stg_pk.at[pl.ds(0, FL)],
                            out_pk.at[pl.ds(wid * CAP + f, FL)])
            pltpu.sync_copy(stg_id.at[pl.ds(0, FL)],
                            out_id.at[pl.ds(wid * CAP + f, FL)])
            sptr[1] = f + FL
            rem = sptr[0] - FL

            def mv(j, _):
                stg_pk[pl.ds(j * 16, 16)] = stg_pk[pl.ds(FL + j * 16, 16)]
                stg_id[pl.ds(j * 16, 16)] = stg_id[pl.ds(FL + j * 16, 16)]
                return 0

            lax.fori_loop(0, (rem + 15) // 16, mv, 0)
            sptr[0] = rem

        return 0

    lax.fori_loop(0, E // CH, chunk_body, 0)

    @pl.when(sptr[0] > 0)
    def _():
        f2 = pl.multiple_of(sptr[1], 128)
        pltpu.sync_copy(stg_pk.at[pl.ds(0, FL)],
                        out_pk.at[pl.ds(wid * CAP + f2, FL)])
        pltpu.sync_copy(stg_id.at[pl.ds(0, FL)],
                        out_id.at[pl.ds(wid * CAP + f2, FL)])

    carry = jnp.int32(0)
    for j in range(NPT // 16):
        hvv = hv[pl.ds(j * 16, 16)]
        cs = plsc.cumsum(hvv)
        ex = cs - hvv + carry
        offv[pl.ds(j * 16, 16)] = ex
        cur[pl.ds(j * 16, 16)] = ex
        carry = carry + cs[15]
    offv[pl.ds(NPT, 16)] = jnp.broadcast_to(carry, (16,))
    pltpu.sync_copy(offv, out_off.at[pl.ds(wid * 512, 512)])

    K = carry

    def place(ci, _):
        b2 = pl.multiple_of(ci * CP, 128)
        pltpu.sync_copy(out_pk.at[pl.ds(wid * CAP + b2, CP)], pkb)
        pltpu.sync_copy(out_id.at[pl.ds(wid * CAP + b2, CP)], idb)

        def pv(i, _):
            posg = b2 + i * 16 + lane
            m = posg < K
            pk = pkb[pl.ds(i * 16, 16)]
            dl = jnp.where(m, pk & 1023, 0)
            bofs = plsc.load_gather(cur, [dl])
            rk, _last = plsc.scan_count(dl, mask=m)
            pos = bofs + rk - 1
            plsc.addupdate_scatter(cur, [dl], ones, mask=m)
            posb[pl.ds(i * 16, 16)] = jnp.where(m, wid * CAP + pos, NW * CAP)
            return 0

        lax.fori_loop(0, CP // 16, pv, 0)
        pltpu.async_copy(pkb, out_spk.at[posb], sem).wait()
        pltpu.async_copy(idb, out_sid.at[posb], sem).wait()
        return 0

    lax.fori_loop(0, (K + CP - 1) // CP, place, 0)

    vtmp[...] = jnp.broadcast_to(K, (16,))
    pltpu.sync_copy(vtmp, out_cnt.at[pl.ds(wid * 128, 16)])


@functools.partial(
    pl.kernel, mesh=_mesh, compiler_params=_sc_params,
    out_type=(jax.ShapeDtypeStruct((E + 8, 32), jnp.float32),
              jax.ShapeDtypeStruct((NW * NPT * HD,), jnp.float32)),
    scratch_types=[pltpu.VMEM((CE,), jnp.int32),
                   pltpu.VMEM((CE,), jnp.int32),
                   pltpu.VMEM((CE,), jnp.int32),
                   pltpu.VMEM((CE,), jnp.int32),
                   pltpu.VMEM((16,), jnp.int32),
                   pltpu.VMEM((CE, 32), jnp.float32),
                   pltpu.VMEM((CE, 32), jnp.float32),
                   pltpu.VMEM((CE, 32), jnp.float32),
                   pltpu.VMEM((NPT * 32,), jnp.float32),
                   pltpu.VMEM((NPT * 32,), jnp.float32),
                   pltpu.VMEM((NPT * 32,), jnp.float32),
                   pltpu.VMEM((512,), jnp.int32),
                   pltpu.VMEM((W,), jnp.int32),
                   pltpu.VMEM((W,), jnp.int32),
                   pltpu.VMEM((2 * W,), jnp.int32),
                   pltpu.VMEM((2 * W,), jnp.int32),
                   pltpu.VMEM((2 * W, HD), jnp.float32),
                   pltpu.VMEM((2 * W, 32), jnp.float32),
                   pltpu.VMEM((32,), jnp.float32),
                   pltpu.VMEM((8 * HD,), jnp.float32),
                   pltpu.SMEM((8,), jnp.int32),
                   pltpu.SemaphoreType.DMA,
                   pltpu.SemaphoreType.DMA,
                   pltpu.SemaphoreType.DMA,
                   pltpu.SemaphoreType.DMA,
                   pltpu.SemaphoreType.DMA],
)
def _edge_phase(spk_hbm, sid_hbm, off_hbm, cnt_hbm, as2d, ad_flat, ae2d, h2d,
                alx, outw,
                pkb, idb, dlb, srcb, vtmpi, asr2, aer2, alb2,
                ad_loc, amax, den, offl, pkc, idc, srcw, idw, hw, exw,
                arow, orow, sp, gsem, hsemA, hsemB, xsemA, xsemB):
    wid = lax.axis_index("s") * NC + lax.axis_index("c")
    lo = wid * NPT
    nn = jnp.minimum(NPT, N - lo)
    lane = lax.iota(jnp.int32, 16)
    ninf = jnp.broadcast_to(jnp.float32(-jnp.inf), (16,))
    fzero = jnp.broadcast_to(jnp.float32(0.0), (16,))

    pltpu.sync_copy(cnt_hbm.at[pl.ds(wid * 128, 16)], vtmpi)
    K = vtmpi[...][0]
    pltpu.sync_copy(off_hbm.at[pl.ds(wid * 512, 512)], offl)
    pltpu.sync_copy(ad_flat.at[pl.ds(wid * (NPT * 32), NPT * 32)], ad_loc)

    def init_body(j, _):
        amax[pl.ds(j * 16, 16)] = ninf
        den[pl.ds(j * 16, 16)] = fzero
        return 0
    lax.fori_loop(0, NPT * 32 // 16, init_body, 0)

    NCH = (K + CE - 1) // CE

    def load_meta(ci):
        b2 = pl.multiple_of(ci * CE, 128)
        pltpu.sync_copy(spk_hbm.at[pl.ds(wid * CAP + b2, CE)], pkb)
        pltpu.sync_copy(sid_hbm.at[pl.ds(wid * CAP + b2, CE)], idb)

        def san(v, _):
            pk = pkb[pl.ds(v * 16, 16)]
            idv = idb[pl.ds(v * 16, 16)]
            val = (b2 + v * 16 + lane) < K
            srcb[pl.ds(v * 16, 16)] = jnp.where(val, pk >> 10, 0)
            dlb[pl.ds(v * 16, 16)] = jnp.where(val, pk & 1023, 0)
            idb[pl.ds(v * 16, 16)] = jnp.where(val, idv, E)
            return 0
        lax.fori_loop(0, CE // 16, san, 0)
        return b2

    def passA(ci, _):
        b2 = load_meta(ci)
        pltpu.async_copy(as2d.at[srcb], asr2, gsem).wait()
        pltpu.async_copy(ae2d.at[idb], aer2, gsem).wait()

        def ebody(e, _):
            eg = b2 + e
            dlv = plsc.load_gather(dlb, [jnp.broadcast_to(e, (16,))])
            mv = jnp.broadcast_to(eg < K, (16,))
            for half in range(2):
                hb = half * 16
                ii = dlv * 32 + hb + lane
                x = (asr2[e, pl.ds(hb, 16)] + aer2[e, pl.ds(hb, 16)]
                     + plsc.load_gather(ad_loc, [ii]))
                al = jnp.maximum(x, 0.2 * x)
                alb2[e, pl.ds(hb, 16)] = al
                m = mv & ((hb + lane) < 20)
                old = plsc.load_gather(amax, [ii])
                plsc.store_scatter(amax, [ii], jnp.maximum(old, al), mask=m)
            return 0
        lax.fori_loop(0, CE, ebody, 0)
        pltpu.async_copy(alb2, alx.at[idb], gsem).wait()
        return 0
    lax.fori_loop(0, NCH, passA, 0)

    def passB(ci, _):
        b2 = load_meta(ci)
        pltpu.async_copy(alx.at[idb], alb2, gsem).wait()

        def ebody(e, _):
            eg = b2 + e
            dlv = plsc.load_gather(dlb, [jnp.broadcast_to(e, (16,))])
            mv = jnp.broadcast_to(eg < K, (16,))
            for half in range(2):
                hb = half * 16
                ii = dlv * 32 + hb + lane
                alv = alb2[e, pl.ds(hb, 16)]
                amx = plsc.load_gather(amax, [ii])
                exv = jnp.exp(alv - amx)
                alb2[e, pl.ds(hb, 16)] = exv
                m = mv & ((hb + lane) < 20)
                plsc.addupdate_scatter(den, [ii], exv, mask=m)
            return 0
        lax.fori_loop(0, CE, ebody, 0)
        pltpu.async_copy(alb2, alx.at[idb], gsem).wait()
        return 0
    lax.fori_loop(0, NCH, passB, 0)

    def fire(wstart, half, hsem, xsem):
        b = pl.multiple_of(wstart, 128)
        pltpu.sync_copy(spk_hbm.at[pl.ds(wid * CAP + b, W)], pkc)
        pltpu.sync_copy(sid_hbm.at[pl.ds(wid * CAP + b, W)], idc)

        def san(v, _):
            pk = pkc[pl.ds(v * 16, 16)]
            idv = idc[pl.ds(v * 16, 16)]
            val = (b + v * 16 + lane) < K
            srcw[pl.ds(half * W + v * 16, 16)] = jnp.where(val, pk >> 10, 0)
            idw[pl.ds(half * W + v * 16, 16)] = jnp.where(val, idv, E)
            return 0
        lax.fori_loop(0, W // 16, san, 0)
        pltpu.async_copy(h2d.at[srcw.at[pl.ds(half * W, W)]],
                         hw.at[pl.ds(half * W, W)], hsem)
        pltpu.async_copy(alx.at[idw.at[pl.ds(half * W, W)]],
                         exw.at[pl.ds(half * W, W)], xsem)

    def wait_w(half, hsem, xsem):
        pltpu.make_async_copy(h2d.at[pl.ds(0, W)],
                              hw.at[pl.ds(half * W, W)], hsem).wait()
        pltpu.make_async_copy(alx.at[pl.ds(0, W)],
                              exw.at[pl.ds(half * W, W)], xsem).wait()

    ihx = []
    for v in range(13):
        f = v * 16 + lane
        hidx = (f * 6554) >> 16
        hidx = jnp.where(f < 200, hidx, 0)
        ihx.append(hidx)

    fire(0, 0, hsemA, xsemA)
    fire(W, 1, hsemB, xsemB)
    wait_w(0, hsemA, xsemA)
    sp[0] = W

    def node_body(n, _):
        ovec = offl[pl.ds(n, 16)]
        s0 = ovec[0]
        s1 = ovec[1]
        d0 = den[pl.ds(n * 32, 16)]
        d1 = den[pl.ds(n * 32 + 16, 16)]
        inv0 = 1.0 / (d0 + 1e-16)
        inv1 = 1.0 / (d1 + 1e-16)

        def ebody(e, acc):
            @pl.when(e >= sp[0])
            def _():
                hk = (sp[0] >> 7) & 1

                @pl.when(hk == 0)
                def _():
                    wait_w(0, hsemA, xsemA)
                    fire(sp[0] + W, 1, hsemB, xsemB)

                @pl.when(hk == 1)
                def _():
                    wait_w(1, hsemB, xsemB)
                    fire(sp[0] + W, 0, hsemA, xsemA)

                sp[0] = sp[0] + W

            r = e & (2 * W - 1)
            ex0 = exw[r, pl.ds(0, 16)]
            ex1 = exw[r, pl.ds(16, 16)]
            arow[pl.ds(0, 16)] = ex0 * inv0
            arow[pl.ds(16, 16)] = ex1 * inv1
            out = []
            for v in range(13):
                hv = hw[r, pl.ds(v * 16, 16)]
                av = plsc.load_gather(arow, [ihx[v]])
                if v == 12:
                    av = jnp.where((v * 16 + lane) < 200, av, 0.0)
                out.append(jnp.maximum(acc[v], hv * av))
            return tuple(out)

        acc0 = tuple(ninf for _ in range(13))
        acc = lax.fori_loop(s0, s1, ebody, acc0)
        ob = (n & 7) * HD
        for v in range(13):
            accv = jnp.where(acc[v] == -jnp.inf, 0.0, acc[v])
            orow[pl.ds(ob + v * 16, 16)] = accv

        @pl.when((n & 7) == 7)
        def _():
            o = pl.multiple_of((wid * NPT + n - 7) * HD, 128)
            pltpu.sync_copy(orow, outw.at[pl.ds(o, 8 * HD)])
        return 0

    lax.fori_loop(0, nn, node_body, 0)

    hk_end = (sp[0] >> 7) & 1

    @pl.when(hk_end == 0)
    def _():
        wait_w(0, hsemA, xsemA)

    @pl.when(hk_end == 1)
    def _():
        wait_w(1, hsemB, xsemB)


def _node_body(x_ref, w_ref, as_ref, ad_ref, h_ref, asn_ref, adn_ref):
    h = jnp.dot(x_ref[...], w_ref[...], preferred_element_type=jnp.float32)
    h_ref[...] = h
    asn_ref[...] = jnp.dot(h, as_ref[...], preferred_element_type=jnp.float32)
    adn_ref[...] = jnp.dot(h, ad_ref[...], preferred_element_type=jnp.float32)


def _node_tc(x, wp, asp, adp, bm):
    m, k = x.shape
    return pl.pallas_call(
        _node_body,
        grid=(m // bm,),
        in_specs=[pl.BlockSpec((bm, k), lambda i: (i, 0)),
                  pl.BlockSpec((k, HD), lambda i: (0, 0)),
                  pl.BlockSpec((HD, 32), lambda i: (0, 0)),
                  pl.BlockSpec((HD, 32), lambda i: (0, 0))],
        out_specs=[pl.BlockSpec((bm, HD), lambda i: (i, 0)),
                   pl.BlockSpec((bm, 32), lambda i: (i, 0)),
                   pl.BlockSpec((bm, 32), lambda i: (i, 0))],
        out_shape=[jax.ShapeDtypeStruct((m, HD), jnp.float32),
                   jax.ShapeDtypeStruct((m, 32), jnp.float32),
                   jax.ShapeDtypeStruct((m, 32), jnp.float32)],
    )(x, wp, asp, adp)


def _mid_body(o1_ref, mm_ref, b_ref, w_ref, as_ref, ad_ref,
              h_ref, asn_ref, adn_ref):
    x2 = jnp.dot(o1_ref[...], mm_ref[...], preferred_element_type=jnp.float32)
    x2 = jnp.maximum(x2 + b_ref[0:1, :], 0.0)
    h = jnp.dot(x2, w_ref[...], preferred_element_type=jnp.float32)
    h_ref[...] = h
    asn_ref[...] = jnp.dot(h, as_ref[...], preferred_element_type=jnp.float32)
    adn_ref[...] = jnp.dot(h, ad_ref[...], preferred_element_type=jnp.float32)


def _mid_tc(o1, mm, bp, wp, asp, adp, bm):
    m = o1.shape[0]
    return pl.pallas_call(
        _mid_body,
        grid=(m // bm,),
        in_specs=[pl.BlockSpec((bm, HD), lambda i: (i, 0)),
                  pl.BlockSpec((HD, 16), lambda i: (0, 0)),
                  pl.BlockSpec((8, 16), lambda i: (0, 0)),
                  pl.BlockSpec((16, HD), lambda i: (0, 0)),
                  pl.BlockSpec((HD, 32), lambda i: (0, 0)),
                  pl.BlockSpec((HD, 32), lambda i: (0, 0))],
        out_specs=[pl.BlockSpec((bm, HD), lambda i: (i, 0)),
                   pl.BlockSpec((bm, 32), lambda i: (i, 0)),
                   pl.BlockSpec((bm, 32), lambda i: (i, 0))],
        out_shape=[jax.ShapeDtypeStruct((m, HD), jnp.float32),
                   jax.ShapeDtypeStruct((m, 32), jnp.float32),
                   jax.ShapeDtypeStruct((m, 32), jnp.float32)],
    )(o1, mm, bp, wp, asp, adp)


def _ae_body(a_ref, b_ref, o1_ref, o2_ref):
    r = jnp.dot(a_ref[...], b_ref[...], preferred_element_type=jnp.float32)
    o1_ref[...] = r[:, :32]
    o2_ref[...] = r[:, 32:]


def _ae_tc(ea, bp, bm):
    m = ea.shape[0]
    return pl.pallas_call(
        _ae_body,
        grid=(m // bm,),
        in_specs=[pl.BlockSpec((bm, D_E), lambda i: (i, 0)),
                  pl.BlockSpec((D_E, 64), lambda i: (0, 0))],
        out_specs=[pl.BlockSpec((bm, 32), lambda i: (i, 0)),
                   pl.BlockSpec((bm, 32), lambda i: (i, 0))],
        out_shape=[jax.ShapeDtypeStruct((m, 32), jnp.float32),
                   jax.ShapeDtypeStruct((m, 32), jnp.float32)],
    )(ea, bp)


def _fin_body(o2_ref, mm_ref, b_ref, o_ref):
    lg = jnp.dot(o2_ref[...], mm_ref[...], preferred_element_type=jnp.float32)
    lg = lg + b_ref[0:1, :]
    col = lax.broadcasted_iota(jnp.int32, lg.shape, 1)
    lg = jnp.where(col < NUM_CLASSES, lg, -jnp.inf)
    mx = jnp.max(lg, axis=-1, keepdims=True)
    z = lg - mx
    s = jnp.sum(jnp.where(col < NUM_CLASSES, jnp.exp(z), 0.0),
                axis=-1, keepdims=True)
    o_ref[...] = (z - jnp.log(s))[:, :NUM_CLASSES]


def _fin_tc(o2, mm, bp, bm):
    m = o2.shape[0]
    return pl.pallas_call(
        _fin_body,
        grid=(m // bm,),
        in_specs=[pl.BlockSpec((bm, HD), lambda i: (i, 0)),
                  pl.BlockSpec((HD, 16), lambda i: (0, 0)),
                  pl.BlockSpec((8, 16), lambda i: (0, 0))],
        out_specs=pl.BlockSpec((bm, NUM_CLASSES), lambda i: (i, 0)),
        out_shape=jax.ShapeDtypeStruct((m, NUM_CLASSES), jnp.float32),
    )(o2, mm, bp)


def _headvec_mat(a):
    idx = jnp.arange(H * C)
    m = jnp.zeros((HD, 32), jnp.float32)
    return m.at[idx, idx // C].set(a.reshape(-1))


def kernel(x, edge_index, edge_attr, W1, as1, ad1, We1, ae1, b1,
           W2, as2, ad2, We2, ae2, b2):
    src = edge_index[0]
    dst = edge_index[1]

    _upk, _uid, spk, sid, off, cnt = _partition(src, dst)

    B1 = (We1.reshape(D_E, H, C) * ae1[None]).sum(-1)
    B2 = (We2.reshape(D_E, H, C) * ae2[None]).sum(-1)
    Bp = jnp.zeros((D_E, 64), jnp.float32)
    Bp = Bp.at[:, :H].set(B1).at[:, 32:32 + H].set(B2)
    ae1p, ae2p = _ae_tc(edge_attr, Bp, 8000)
    ae1p = jnp.pad(ae1p, ((0, 8), (0, 0)))
    ae2p = jnp.pad(ae2p, ((0, 8), (0, 0)))

    W1p = jnp.pad(W1, ((0, 0), (0, HD - H * C)))
    h1, as1n, ad1n = _node_tc(x, W1p, _headvec_mat(as1), _headvec_mat(ad1),
                              1000)
    ad1f = jnp.pad(ad1n, ((0, NW * NPT - N), (0, 0))).reshape(-1)
    _alx1, out1w = _edge_phase(spk, sid, off, cnt, as1n, ad1f, ae1p, h1)
    out1 = out1w.reshape(NW * NPT, HD)[:N]

    mm = jnp.zeros((HD, 16), jnp.float32)
    mm = mm.at[jnp.arange(H * C), jnp.arange(H * C) % C].set(1.0 / H)
    b1p = jnp.zeros((8, 16), jnp.float32).at[0, :C].set(b1)
    b2p = jnp.zeros((8, 16), jnp.float32).at[0, :C].set(b2)
    W2p = jnp.pad(W2, ((0, 16 - C), (0, HD - H * C)))
    h2, as2n, ad2n = _mid_tc(out1, mm, b1p, W2p,
                             _headvec_mat(as2), _headvec_mat(ad2), 1000)
    ad2f = jnp.pad(ad2n, ((0, NW * NPT - N), (0, 0))).reshape(-1)
    _alx2, out2w = _edge_phase(spk, sid, off, cnt, as2n, ad2f, ae2p, h2)
    out2 = out2w.reshape(NW * NPT, HD)[:N]

    return _fin_tc(out2, mm, b2p, 1000)
